# baseline (speedup 1.0000x reference)
import functools

import jax
import jax.numpy as jnp
from jax import lax
from jax.experimental import pallas as pl
from jax.experimental.pallas import tpu as pltpu
from jax.experimental.pallas import tpu_sc as plsc

_EPS = 1e-07
_ALPHA = 0.5

_NC = 2
_NS = 16
_NW = _NC * _NS
_L = 16


def _spline_tables(cps0):
    cps = jnp.concatenate([cps0, cps0[0:1, :]], axis=0)
    l01 = jnp.sqrt(jnp.sum(jnp.power(cps[0, :] - cps[1, :], 2)) + _EPS)
    l_last = jnp.sqrt(jnp.sum(jnp.power(cps[-1, :] - cps[-2, :], 2)) + _EPS)
    first = cps[0, :] - l01 / l_last * (cps[-1, :] - cps[-2, :])
    last = cps[-1, :] + l_last / l01 * (cps[1, :] - cps[0, :])
    aux = jnp.concatenate([first[None, :], cps, last[None, :]], axis=0)
    d = jnp.power(jnp.sum(jnp.power(aux[1:] - aux[:-1], 2), axis=-1),
                  _ALPHA / 2.0)
    tk = jnp.concatenate([jnp.zeros(1, dtype=jnp.float32), jnp.cumsum(d)])
    return aux, tk


def _poly_lerp(p, q, ta, tb):
    r = 1.0 / (tb - ta)
    return [(tb * p[i] - (p[i - 1] if i else 0.0)
             + (q[i - 1] if i else 0.0) - ta * q[i]) * r for i in range(4)]


def _segment_cubic(aux, tk, s, d):
    t = [tk[s - 1 + i] for i in range(4)]
    a = [[aux[s - 1 + i, d], 0.0, 0.0, 0.0] for i in range(4)]
    x01 = _poly_lerp(a[0], a[1], t[0], t[1])
    x12 = _poly_lerp(a[1], a[2], t[1], t[2])
    x23 = _poly_lerp(a[2], a[3], t[2], t[3])
    x012 = _poly_lerp(x01, x12, t[0], t[2])
    x123 = _poly_lerp(x12, x23, t[1], t[3])
    return _poly_lerp(x012, x123, t[1], t[2])


def _make_sc_eval(n, chunk):
    nvec = chunk // _L
    mesh = plsc.VectorSubcoreMesh(core_axis_name="c", subcore_axis_name="s",
                                  num_cores=_NC, num_subcores=_NS)

    @functools.partial(
        pl.kernel,
        out_type=jax.ShapeDtypeStruct((2 * n,), jnp.float32),
        mesh=mesh,
        compiler_params=pltpu.CompilerParams(needs_layout_passes=False),
        scratch_types=[
            pltpu.VMEM((chunk,), jnp.float32),
            pltpu.VMEM((2 * chunk,), jnp.float32),
            pltpu.VMEM((17 * _L,), jnp.float32),
        ],
    )
    def spline_eval(t_hbm, c_hbm, out_hbm, tbuf, obuf, cbuf):
        wid = lax.axis_index("s") * _NC + lax.axis_index("c")
        base = jnp.minimum(wid * chunk, n - chunk)
        pltpu.sync_copy(t_hbm.at[pl.ds(base, chunk)], tbuf)
        pltpu.sync_copy(c_hbm, cbuf)

        c = [cbuf[pl.ds(_L * k, _L)] for k in range(17)]
        tk2 = c[0]
        iot2 = lax.iota(jnp.int32, _L) * 2

        def step(j, carry):
            tv = tbuf[pl.ds(j * _L, _L)]
            m = tv >= tk2
            idx = iot2 + j * (2 * _L)
            for d in range(2):
                o1, o2 = 1 + 4 * d, 9 + 4 * d
                cc = [jnp.where(m, c[o2 + i], c[o1 + i]) for i in range(4)]
                p = ((cc[3] * tv + cc[2]) * tv + cc[1]) * tv + cc[0]
                plsc.store_scatter(obuf, [idx + d], p)
            return carry

        lax.fori_loop(0, nvec, step, 0)
        pltpu.sync_copy(obuf, out_hbm.at[pl.ds(2 * base, 2 * chunk)])

    return spline_eval


def kernel(t, cps):
    n = t.shape[0]
    aux, tk = _spline_tables(cps)

    rows = [tk[2]]
    for s in (1, 2):
        for d in (0, 1):
            rows += list(_segment_cubic(aux, tk, s, d))
    consts = jnp.stack(rows).astype(jnp.float32)
    cvec = jnp.broadcast_to(consts[:, None], (17, _L)).reshape(-1)

    assert n % 8 == 0
    gran = _NW * _L
    chunk = ((n + gran - 1) // gran) * _L
    flat = _make_sc_eval(n, chunk)(t, cvec)
    return flat.reshape(n, 2)


# speedup vs baseline: 1.9770x; 1.9770x over previous
import functools

import jax
import jax.numpy as jnp
from jax import lax
from jax.experimental import pallas as pl
from jax.experimental.pallas import tpu as pltpu
from jax.experimental.pallas import tpu_sc as plsc

_EPS = 1e-07

_NC = 2
_NS = 16
_NW = _NC * _NS
_L = 16


def _poly_lerp(p, q, ta, tb):
    r = 1.0 / (tb - ta)
    return [(tb * p[i] - (p[i - 1] if i else 0.0)
             + (q[i - 1] if i else 0.0) - ta * q[i]) * r for i in range(4)]


def _segment_cubic(aux, tk, s, d):
    t = [tk[s - 1 + i] for i in range(4)]
    a = [[aux[s - 1 + i][d], 0.0, 0.0, 0.0] for i in range(4)]
    x01 = _poly_lerp(a[0], a[1], t[0], t[1])
    x12 = _poly_lerp(a[1], a[2], t[1], t[2])
    x23 = _poly_lerp(a[2], a[3], t[2], t[3])
    x012 = _poly_lerp(x01, x12, t[0], t[2])
    x123 = _poly_lerp(x12, x23, t[1], t[3])
    return _poly_lerp(x012, x123, t[1], t[2])


def _setup_body(cps_ref, out_ref):
    p0 = (cps_ref[0, 0], cps_ref[0, 1])
    p1 = (cps_ref[1, 0], cps_ref[1, 1])
    dx, dy = p0[0] - p1[0], p0[1] - p1[1]
    l01 = jnp.sqrt(dx * dx + dy * dy + _EPS)
    l_last = jnp.sqrt(dx * dx + dy * dy + _EPS)
    ra = l01 / l_last
    rb = l_last / l01
    first = (p0[0] - ra * dx, p0[1] - ra * dy)
    last = (p0[0] + rb * (p1[0] - p0[0]), p0[1] + rb * (p1[1] - p0[1]))
    aux = [first, p0, p1, p0, last]

    tk = [jnp.float32(0.0)]
    for i in range(4):
        ddx = aux[i + 1][0] - aux[i][0]
        ddy = aux[i + 1][1] - aux[i][1]
        tk.append(tk[-1] + jnp.sqrt(jnp.sqrt(ddx * ddx + ddy * ddy)))

    out_ref[0] = tk[2]
    for z in range(17, 32):
        out_ref[z] = jnp.float32(0.0)
    k = 1
    for s in (1, 2):
        for d in (0, 1):
            for coef in _segment_cubic(aux, tk, s, d):
                out_ref[k] = jnp.float32(coef)
                k += 1


def _spline_scalars(cps):
    return pl.pallas_call(
        _setup_body,
        out_shape=jax.ShapeDtypeStruct((32,), jnp.float32),
        in_specs=[pl.BlockSpec(memory_space=pltpu.SMEM)],
        out_specs=pl.BlockSpec(memory_space=pltpu.SMEM),
    )(cps)


def _make_sc_eval(n, chunk):
    nvec = chunk // _L
    mesh = plsc.VectorSubcoreMesh(core_axis_name="c", subcore_axis_name="s",
                                  num_cores=_NC, num_subcores=_NS)

    @functools.partial(
        pl.kernel,
        out_type=jax.ShapeDtypeStruct((2 * n,), jnp.float32),
        mesh=mesh,
        compiler_params=pltpu.CompilerParams(needs_layout_passes=False),
        scratch_types=[
            pltpu.VMEM((chunk,), jnp.float32),
            pltpu.VMEM((2 * chunk,), jnp.float32),
            pltpu.VMEM((32 * _L,), jnp.float32),
        ],
    )
    def spline_eval(t_hbm, c_hbm, out_hbm, tbuf, obuf, cbuf):
        wid = lax.axis_index("s") * _NC + lax.axis_index("c")
        base = jnp.minimum(wid * chunk, n - chunk)
        pltpu.sync_copy(t_hbm.at[pl.ds(base, chunk)], tbuf)
        pltpu.sync_copy(c_hbm, cbuf)

        c = [cbuf[pl.ds(_L * k, _L)] for k in range(17)]
        tk2 = c[0]
        iot2 = lax.iota(jnp.int32, _L) * 2

        def step(j, carry):
            tv = tbuf[pl.ds(j * _L, _L)]
            m = tv >= tk2
            idx = iot2 + j * (2 * _L)
            for d in range(2):
                o1, o2 = 1 + 4 * d, 9 + 4 * d
                cc = [jnp.where(m, c[o2 + i], c[o1 + i]) for i in range(4)]
                p = ((cc[3] * tv + cc[2]) * tv + cc[1]) * tv + cc[0]
                plsc.store_scatter(obuf, [idx + d], p)
            return carry

        lax.fori_loop(0, nvec, step, 0)
        pltpu.sync_copy(obuf, out_hbm.at[pl.ds(2 * base, 2 * chunk)])

    return spline_eval


def kernel(t, cps):
    n = t.shape[0]
    consts = _spline_scalars(cps)
    consts = jnp.broadcast_to(consts[:, None], (32, _L)).reshape(-1)
    assert n % 8 == 0
    gran = _NW * _L
    chunk = ((n + gran - 1) // gran) * _L
    return _make_sc_eval(n, chunk)(t, consts).reshape(n, 2)


# speedup vs baseline: 2.3380x; 1.1826x over previous
import functools

import jax
import jax.numpy as jnp
from jax import lax
from jax.experimental import pallas as pl
from jax.experimental.pallas import tpu as pltpu
from jax.experimental.pallas import tpu_sc as plsc

_EPS = 1e-07

_NC = 2
_NS = 16
_NW = _NC * _NS
_L = 16


def _poly_lerp(p, q, ta, tb):
    r = 1.0 / (tb - ta)
    return [(tb * p[i] - (p[i - 1] if i else 0.0)
             + (q[i - 1] if i else 0.0) - ta * q[i]) * r for i in range(4)]


def _segment_cubic(aux, tk, s, d):
    t = [tk[s - 1 + i] for i in range(4)]
    a = [[aux[s - 1 + i][d], 0.0, 0.0, 0.0] for i in range(4)]
    x01 = _poly_lerp(a[0], a[1], t[0], t[1])
    x12 = _poly_lerp(a[1], a[2], t[1], t[2])
    x23 = _poly_lerp(a[2], a[3], t[2], t[3])
    x012 = _poly_lerp(x01, x12, t[0], t[2])
    x123 = _poly_lerp(x12, x23, t[1], t[3])
    return _poly_lerp(x012, x123, t[1], t[2])


def _setup_body(cps_ref, out_ref):
    p0 = (cps_ref[0, 0], cps_ref[0, 1])
    p1 = (cps_ref[1, 0], cps_ref[1, 1])
    dx, dy = p0[0] - p1[0], p0[1] - p1[1]
    l01 = jnp.sqrt(dx * dx + dy * dy + _EPS)
    l_last = jnp.sqrt(dx * dx + dy * dy + _EPS)
    ra = l01 / l_last
    rb = l_last / l01
    first = (p0[0] - ra * dx, p0[1] - ra * dy)
    last = (p0[0] + rb * (p1[0] - p0[0]), p0[1] + rb * (p1[1] - p0[1]))
    aux = [first, p0, p1, p0, last]

    tk = [jnp.float32(0.0)]
    for i in range(4):
        ddx = aux[i + 1][0] - aux[i][0]
        ddy = aux[i + 1][1] - aux[i][1]
        tk.append(tk[-1] + jnp.sqrt(jnp.sqrt(ddx * ddx + ddy * ddy)))

    out_ref[0] = tk[2]
    for z in range(17, 32):
        out_ref[z] = jnp.float32(0.0)
    k = 1
    for s in (1, 2):
        for d in (0, 1):
            for coef in _segment_cubic(aux, tk, s, d):
                out_ref[k] = jnp.float32(coef)
                k += 1


def _spline_scalars(cps):
    return pl.pallas_call(
        _setup_body,
        out_shape=jax.ShapeDtypeStruct((32,), jnp.float32),
        in_specs=[pl.BlockSpec(memory_space=pltpu.SMEM)],
        out_specs=pl.BlockSpec(memory_space=pltpu.SMEM),
    )(cps)


def _make_sc_eval(n, chunk):
    nvec = chunk // _L
    mesh = plsc.VectorSubcoreMesh(core_axis_name="c", subcore_axis_name="s",
                                  num_cores=_NC, num_subcores=_NS)

    @functools.partial(
        pl.kernel,
        out_type=jax.ShapeDtypeStruct((n, 2), jnp.float32),
        mesh=mesh,
        compiler_params=pltpu.CompilerParams(needs_layout_passes=False,
                                            use_tc_tiling_on_sc=False),
        scratch_types=[
            pltpu.VMEM((chunk,), jnp.float32),
            pltpu.VMEM((chunk, 2), jnp.float32),
            pltpu.VMEM((32 * _L,), jnp.float32),
        ],
    )
    def spline_eval(t_hbm, c_hbm, out_hbm, tbuf, obuf, cbuf):
        wid = lax.axis_index("s") * _NC + lax.axis_index("c")
        base = jnp.minimum(wid * chunk, n - chunk)
        pltpu.sync_copy(t_hbm.at[pl.ds(base, chunk)], tbuf)
        pltpu.sync_copy(c_hbm, cbuf)

        c = [cbuf[pl.ds(_L * k, _L)] for k in range(17)]
        tk2 = c[0]
        iota = lax.iota(jnp.int32, _L)
        col = [jnp.full((_L,), d, jnp.int32) for d in (0, 1)]

        def step(j, carry):
            tv = tbuf[pl.ds(j * _L, _L)]
            m = tv >= tk2
            row = iota + j * _L
            for d in range(2):
                o1, o2 = 1 + 4 * d, 9 + 4 * d
                cc = [jnp.where(m, c[o2 + i], c[o1 + i]) for i in range(4)]
                p = ((cc[3] * tv + cc[2]) * tv + cc[1]) * tv + cc[0]
                plsc.store_scatter(obuf, [row, col[d]], p)
            return carry

        lax.fori_loop(0, nvec, step, 0)
        pltpu.sync_copy(obuf, out_hbm.at[pl.ds(base, chunk)])

    return spline_eval


def kernel(t, cps):
    n = t.shape[0]
    consts = _spline_scalars(cps)
    consts = jnp.broadcast_to(consts[:, None], (32, _L)).reshape(-1)
    assert n % 8 == 0
    gran = _NW * _L
    chunk = ((n + gran - 1) // gran) * _L
    return _make_sc_eval(n, chunk)(t, consts)


# speedup vs baseline: 5.1403x; 2.1986x over previous
import functools

import jax
import jax.numpy as jnp
from jax import lax
from jax.experimental import pallas as pl
from jax.experimental.pallas import tpu as pltpu
from jax.experimental.pallas import tpu_sc as plsc

_EPS = 1e-07

_NC = 2
_NS = 16
_NW = _NC * _NS
_L = 16


def _poly_lerp(p, q, ta, tb):
    r = 1.0 / (tb - ta)
    return [(tb * p[i] - (p[i - 1] if i else 0.0)
             + (q[i - 1] if i else 0.0) - ta * q[i]) * r for i in range(4)]


def _segment_cubic(aux, tk, s, d):
    t = [tk[s - 1 + i] for i in range(4)]
    a = [[aux[s - 1 + i][d], 0.0, 0.0, 0.0] for i in range(4)]
    x01 = _poly_lerp(a[0], a[1], t[0], t[1])
    x12 = _poly_lerp(a[1], a[2], t[1], t[2])
    x23 = _poly_lerp(a[2], a[3], t[2], t[3])
    x012 = _poly_lerp(x01, x12, t[0], t[2])
    x123 = _poly_lerp(x12, x23, t[1], t[3])
    return _poly_lerp(x012, x123, t[1], t[2])


def _setup_body(cps_ref, out_ref):
    p0 = (cps_ref[0, 0], cps_ref[0, 1])
    p1 = (cps_ref[1, 0], cps_ref[1, 1])
    dx, dy = p0[0] - p1[0], p0[1] - p1[1]
    l01 = jnp.sqrt(dx * dx + dy * dy + _EPS)
    l_last = jnp.sqrt(dx * dx + dy * dy + _EPS)
    ra = l01 / l_last
    rb = l_last / l01
    first = (p0[0] - ra * dx, p0[1] - ra * dy)
    last = (p0[0] + rb * (p1[0] - p0[0]), p0[1] + rb * (p1[1] - p0[1]))
    aux = [first, p0, p1, p0, last]

    tk = [jnp.float32(0.0)]
    for i in range(4):
        ddx = aux[i + 1][0] - aux[i][0]
        ddy = aux[i + 1][1] - aux[i][1]
        tk.append(tk[-1] + jnp.sqrt(jnp.sqrt(ddx * ddx + ddy * ddy)))

    out_ref[0] = tk[2]
    for z in range(17, 32):
        out_ref[z] = jnp.float32(0.0)
    k = 1
    for s in (1, 2):
        for d in (0, 1):
            for coef in _segment_cubic(aux, tk, s, d):
                out_ref[k] = jnp.float32(coef)
                k += 1


def _spline_scalars(cps):
    return pl.pallas_call(
        _setup_body,
        out_shape=jax.ShapeDtypeStruct((32,), jnp.float32),
        in_specs=[pl.BlockSpec(memory_space=pltpu.SMEM)],
        out_specs=pl.BlockSpec(memory_space=pltpu.SMEM),
    )(cps)


def _make_sc_eval(n, chunk):
    nvec = chunk // _L
    mesh = plsc.VectorSubcoreMesh(core_axis_name="c", subcore_axis_name="s",
                                  num_cores=_NC, num_subcores=_NS)

    @functools.partial(
        pl.kernel,
        out_type=jax.ShapeDtypeStruct((2, n), jnp.float32),
        mesh=mesh,
        compiler_params=pltpu.CompilerParams(needs_layout_passes=False,
                                            use_tc_tiling_on_sc=False),
        scratch_types=[
            pltpu.VMEM((chunk,), jnp.float32),
            pltpu.VMEM((2, chunk), jnp.float32),
            pltpu.VMEM((32 * _L,), jnp.float32),
        ],
    )
    def spline_eval(t_hbm, c_hbm, out_hbm, tbuf, obuf, cbuf):
        wid = lax.axis_index("s") * _NC + lax.axis_index("c")
        base = jnp.minimum(wid * chunk, n - chunk)
        pltpu.sync_copy(t_hbm.at[pl.ds(base, chunk)], tbuf)
        pltpu.sync_copy(c_hbm, cbuf)

        c = [cbuf[pl.ds(_L * k, _L)] for k in range(17)]
        tk2 = c[0]
        def step(j, carry):
            tv = tbuf[pl.ds(j * _L, _L)]
            m = tv >= tk2
            for d in range(2):
                o1, o2 = 1 + 4 * d, 9 + 4 * d
                cc = [jnp.where(m, c[o2 + i], c[o1 + i]) for i in range(4)]
                p = ((cc[3] * tv + cc[2]) * tv + cc[1]) * tv + cc[0]
                obuf[d, pl.ds(j * _L, _L)] = p
            return carry

        lax.fori_loop(0, nvec, step, 0)
        pltpu.sync_copy(obuf.at[0, pl.ds(0, chunk)],
                        out_hbm.at[0, pl.ds(base, chunk)])
        pltpu.sync_copy(obuf.at[1, pl.ds(0, chunk)],
                        out_hbm.at[1, pl.ds(base, chunk)])

    return spline_eval


_ROWS_BLK = 2048


def _interleave_body(in_ref, out_ref):
    out_ref[...] = in_ref[...].reshape(_ROWS_BLK, 2)


def _to_pairs(flat, n):
    return pl.pallas_call(
        _interleave_body,
        grid=((n + _ROWS_BLK - 1) // _ROWS_BLK,),
        in_specs=[pl.BlockSpec((2 * _ROWS_BLK,), lambda g: (g,))],
        out_specs=pl.BlockSpec((_ROWS_BLK, 2), lambda g: (g, 0)),
        out_shape=jax.ShapeDtypeStruct((n, 2), jnp.float32),
    )(flat)


def kernel(t, cps):
    n = t.shape[0]
    consts = _spline_scalars(cps)
    consts = jnp.broadcast_to(consts[:, None], (32, _L)).reshape(-1)
    assert n % 8 == 0
    gran = _NW * _L
    chunk = ((n + gran - 1) // gran) * _L
    planes = _make_sc_eval(n, chunk)(t, consts)
    return jnp.stack([planes[0], planes[1]], axis=-1)
